# baseline (speedup 1.0000x reference)
import jax
import jax.numpy as jnp
from jax import lax
from jax.experimental import pallas as pl
from jax.experimental.pallas import tpu as pltpu


_BLOCK_TOKENS = 2048
_ISSUE_UNROLL = 32
_CHUNK_ROWS = 256


def _gather_kernel_body(tokens_per_block, unroll, chunk_rows):
    n_chunks = tokens_per_block // chunk_rows
    forys_per_chunk = chunk_rows // unroll

    def body(ids_ref, table_hbm, out_hbm, rows_vmem, gather_sems, write_sem):
        base = pl.program_id(0) * tokens_per_block

        def issue_chunk(c, carry):
            row = c * unroll
            sem_i = lax.div(c, forys_per_chunk)
            toks = [ids_ref[base + row + u] for u in range(unroll)]
            for u in range(unroll):
                pltpu.make_async_copy(table_hbm.at[pl.ds(toks[u], 1), :],
                                      rows_vmem.at[pl.ds(row + u, 1), :],
                                      gather_sems.at[sem_i]).start()
            return carry

        lax.fori_loop(0, tokens_per_block // unroll, issue_chunk, 0)

        for k in range(n_chunks):
            r0 = k * chunk_rows
            pltpu.make_async_copy(
                table_hbm.at[pl.ds(0, chunk_rows), :],
                rows_vmem.at[pl.ds(r0, chunk_rows), :],
                gather_sems.at[k]).wait()
            pltpu.make_async_copy(
                rows_vmem.at[pl.ds(r0, chunk_rows), :],
                out_hbm.at[pl.ds(base + r0, chunk_rows), :],
                write_sem).start()

        pltpu.make_async_copy(
            rows_vmem.at[pl.ds(0, tokens_per_block), :],
            out_hbm.at[pl.ds(base, tokens_per_block), :],
            write_sem).wait()
    return body


def kernel(x, table):
    b, s = x.shape
    v, d = table.shape
    n = b * s
    dtype = table.dtype
    itemsize = jnp.dtype(dtype).itemsize

    t = min(_BLOCK_TOKENS, n)
    flat_ids = x.reshape(n).astype(jnp.int32)

    cost = pl.CostEstimate(
        flops=0, transcendentals=0,
        bytes_accessed=2 * n * d * itemsize + n * 4)

    out_flat = pl.pallas_call(
        _gather_kernel_body(t, _ISSUE_UNROLL, _CHUNK_ROWS),
        out_shape=jax.ShapeDtypeStruct((n, d), dtype),
        grid_spec=pltpu.PrefetchScalarGridSpec(
            num_scalar_prefetch=1,
            grid=(n // t,),
            in_specs=[pl.BlockSpec(memory_space=pl.ANY)],
            out_specs=pl.BlockSpec(memory_space=pl.ANY),
            scratch_shapes=[
                pltpu.VMEM((t, d), dtype),
                pltpu.SemaphoreType.DMA((t // _CHUNK_ROWS,)),
                pltpu.SemaphoreType.DMA,
            ],
        ),
        compiler_params=pltpu.CompilerParams(
            dimension_semantics=("parallel",),
            disable_bounds_checks=True),
        cost_estimate=cost,
    )(flat_ids, table)
    return out_flat.reshape(b, s, d)


# speedup vs baseline: 1.0047x; 1.0047x over previous
import jax
import jax.numpy as jnp
from jax import lax
from jax.experimental import pallas as pl
from jax.experimental.pallas import tpu as pltpu


_BLOCK_TOKENS = 2048
_ISSUE_UNROLL = 32
_CHUNK_ROWS = 256


def _gather_kernel_body(tokens_per_block, unroll, chunk_rows):
    n_chunks = tokens_per_block // chunk_rows
    forys_per_chunk = chunk_rows // unroll

    def body(ids_ref, table_hbm, out_hbm, rows_vmem, gather_sems, write_sem):
        base = pl.program_id(0) * tokens_per_block

        def issue_chunk(c, carry):
            row = c * unroll
            sem_i = lax.div(c, forys_per_chunk)
            toks = [ids_ref[base + row + u] for u in range(unroll)]
            for u in range(unroll):
                pltpu.make_async_copy(table_hbm.at[pl.ds(toks[u], 1), :],
                                      rows_vmem.at[pl.ds(row + u, 1), :],
                                      gather_sems.at[sem_i]).start()
            return carry

        lax.fori_loop(0, tokens_per_block // unroll, issue_chunk, 0)

        for k in range(n_chunks):
            r0 = k * chunk_rows
            pltpu.make_async_copy(
                table_hbm.at[pl.ds(0, chunk_rows), :],
                rows_vmem.at[pl.ds(r0, chunk_rows), :],
                gather_sems.at[k]).wait()
            pltpu.make_async_copy(
                rows_vmem.at[pl.ds(r0, chunk_rows), :],
                out_hbm.at[pl.ds(base + r0, chunk_rows), :],
                write_sem).start(priority=1)

        pltpu.make_async_copy(
            rows_vmem.at[pl.ds(0, tokens_per_block), :],
            out_hbm.at[pl.ds(base, tokens_per_block), :],
            write_sem).wait()
    return body


def kernel(x, table):
    b, s = x.shape
    v, d = table.shape
    n = b * s
    dtype = table.dtype
    itemsize = jnp.dtype(dtype).itemsize

    t = min(_BLOCK_TOKENS, n)
    flat_ids = x.reshape(n).astype(jnp.int32)

    cost = pl.CostEstimate(
        flops=0, transcendentals=0,
        bytes_accessed=2 * n * d * itemsize + n * 4)

    out_flat = pl.pallas_call(
        _gather_kernel_body(t, _ISSUE_UNROLL, _CHUNK_ROWS),
        out_shape=jax.ShapeDtypeStruct((n, d), dtype),
        grid_spec=pltpu.PrefetchScalarGridSpec(
            num_scalar_prefetch=1,
            grid=(n // t,),
            in_specs=[pl.BlockSpec(memory_space=pl.ANY)],
            out_specs=pl.BlockSpec(memory_space=pl.ANY),
            scratch_shapes=[
                pltpu.VMEM((t, d), dtype),
                pltpu.SemaphoreType.DMA((t // _CHUNK_ROWS,)),
                pltpu.SemaphoreType.DMA,
            ],
        ),
        compiler_params=pltpu.CompilerParams(
            dimension_semantics=("parallel",),
            disable_bounds_checks=True),
        cost_estimate=cost,
    )(flat_ids, table)
    return out_flat.reshape(b, s, d)


# speedup vs baseline: 1.0697x; 1.0647x over previous
import jax
import jax.numpy as jnp
from jax import lax
from jax.experimental import pallas as pl
from jax.experimental.pallas import tpu as pltpu


_BLOCK_TOKENS = 2048
_ISSUE_UNROLL = 64


def _gather_kernel_body(tokens_per_block, unroll):
    def body(ids_ref, table_hbm, out_ref, sem):
        base = pl.program_id(0) * tokens_per_block

        def issue_chunk(c, carry):
            row = c * unroll
            toks = [ids_ref[base + row + u] for u in range(unroll)]
            for u in range(unroll):
                pltpu.make_async_copy(table_hbm.at[pl.ds(toks[u], 1), :],
                                      out_ref.at[pl.ds(row + u, 1), :],
                                      sem).start()
            return carry

        lax.fori_loop(0, tokens_per_block // unroll, issue_chunk, 0)

        pltpu.make_async_copy(table_hbm.at[pl.ds(0, tokens_per_block), :],
                              out_ref.at[pl.ds(0, tokens_per_block), :],
                              sem).wait()
    return body


def kernel(x, table):
    b, s = x.shape
    v, d = table.shape
    n = b * s
    dtype = table.dtype
    itemsize = jnp.dtype(dtype).itemsize

    t = min(_BLOCK_TOKENS, n)
    flat_ids = x.reshape(n).astype(jnp.int32)

    cost = pl.CostEstimate(
        flops=0, transcendentals=0,
        bytes_accessed=2 * n * d * itemsize + n * 4)

    out_flat = pl.pallas_call(
        _gather_kernel_body(t, _ISSUE_UNROLL),
        out_shape=jax.ShapeDtypeStruct((n, d), dtype),
        grid_spec=pltpu.PrefetchScalarGridSpec(
            num_scalar_prefetch=1,
            grid=(n // t,),
            in_specs=[pl.BlockSpec(memory_space=pl.ANY)],
            out_specs=pl.BlockSpec((t, d), lambda i, ids: (i, 0)),
            scratch_shapes=[pltpu.SemaphoreType.DMA],
        ),
        compiler_params=pltpu.CompilerParams(
            dimension_semantics=("parallel",),
            disable_bounds_checks=True),
        cost_estimate=cost,
    )(flat_ids, table)
    return out_flat.reshape(b, s, d)
